# baseline (speedup 1.0000x reference)
import functools

import jax
import jax.numpy as jnp
from jax import lax
from jax.experimental import pallas as pl
from jax.experimental.pallas import tpu as pltpu
from jax.experimental.pallas import tpu_sc as plsc

_info = plsc.get_sparse_core_info()
_NC, _NS, _L = _info.num_cores, _info.num_subcores, _info.num_lanes
_NW = _NC * _NS

_B = 16384
_D = 64
_BPW = _B // _NW
_CHUNK = 128
_NCHUNK = _BPW // _CHUNK

_mesh = plsc.VectorSubcoreMesh(core_axis_name="c", subcore_axis_name="s")


@functools.partial(
    pl.kernel,
    out_type=jax.ShapeDtypeStruct((_B,), jnp.float32),
    mesh=_mesh,
    compiler_params=pltpu.CompilerParams(
        needs_layout_passes=False, use_tc_tiling_on_sc=False
    ),
    scratch_types=[
        pltpu.VMEM((_NCHUNK, _CHUNK), jnp.int32),
        pltpu.VMEM((_NCHUNK, _CHUNK), jnp.int32),
        pltpu.VMEM((_BPW, _D), jnp.float32),
        pltpu.VMEM((_BPW, _D), jnp.float32),
        pltpu.VMEM((_BPW,), jnp.float32),
        pltpu.SemaphoreType.DMA,
    ],
)
def _mf_kernel(user_hbm, item_hbm, ut_hbm, it_hbm, out_hbm,
               uidx, iidx, urows, irows, outv, sem):
    wid = lax.axis_index("s") * _NC + lax.axis_index("c")
    base = wid * _BPW

    pltpu.sync_copy(user_hbm.at[wid], uidx)
    pltpu.sync_copy(item_hbm.at[wid], iidx)

    copies = []
    for j in range(_NCHUNK):
        dst = pl.ds(j * _CHUNK, _CHUNK)
        copies.append(pltpu.async_copy(ut_hbm.at[uidx.at[j]], urows.at[dst], sem))
        copies.append(pltpu.async_copy(it_hbm.at[iidx.at[j]], irows.at[dst], sem))
    for c in copies:
        c.wait()

    lanes = lax.iota(jnp.int32, _L)

    def group(g, carry):
        rows = g * _L + lanes
        acc = jnp.zeros((_L,), jnp.float32)
        for c in range(_D):
            cols = jnp.full((_L,), c, jnp.int32)
            ug = plsc.load_gather(urows, [rows, cols])
            vg = plsc.load_gather(irows, [rows, cols])
            acc = acc + ug * vg
        outv[pl.ds(g * _L, _L)] = acc
        return carry

    lax.fori_loop(0, _BPW // _L, group, 0)

    pltpu.sync_copy(outv, out_hbm.at[pl.ds(base, _BPW)])


def kernel(user, item, user_table, item_table):
    user_r = user.reshape(_NW, _NCHUNK, _CHUNK)
    item_r = item.reshape(_NW, _NCHUNK, _CHUNK)
    return _mf_kernel(user_r, item_r, user_table, item_table)


# speedup vs baseline: 1.5570x; 1.5570x over previous
import functools

import jax
import jax.numpy as jnp
from jax import lax
from jax.experimental import pallas as pl
from jax.experimental.pallas import tpu as pltpu
from jax.experimental.pallas import tpu_sc as plsc

_info = plsc.get_sparse_core_info()
_NC, _NS, _L = _info.num_cores, _info.num_subcores, _info.num_lanes
_NW = _NC * _NS

_B = 16384
_D = 64
_BPW = _B // _NW
_P = 256
_NPASS = _BPW // _P

_mesh = plsc.VectorSubcoreMesh(core_axis_name="c", subcore_axis_name="s")


@functools.partial(
    pl.kernel,
    out_type=jax.ShapeDtypeStruct((_B,), jnp.float32),
    mesh=_mesh,
    compiler_params=pltpu.CompilerParams(needs_layout_passes=False),
    scratch_types=[
        pltpu.VMEM((_BPW,), jnp.int32),
        pltpu.VMEM((_BPW,), jnp.int32),
        pltpu.VMEM((_P, _D), jnp.float32),
        pltpu.VMEM((_P, _D), jnp.float32),
        pltpu.VMEM((_BPW,), jnp.float32),
        pltpu.SemaphoreType.DMA,
    ],
)
def _mf_kernel(user_hbm, item_hbm, ut_hbm, it_hbm, out_hbm,
               uidxv, iidxv, urows, irows, outv, sem):
    wid = lax.axis_index("s") * _NC + lax.axis_index("c")
    base = wid * _BPW

    pltpu.sync_copy(user_hbm.at[pl.ds(base, _BPW)], uidxv)
    pltpu.sync_copy(item_hbm.at[pl.ds(base, _BPW)], iidxv)

    lanes = lax.iota(jnp.int32, _L)

    for p in range(_NPASS):
        p0 = p * _P

        def fire(g, carry):
            uvec = uidxv[pl.ds(p0 + g * _L, _L)]
            ivec = iidxv[pl.ds(p0 + g * _L, _L)]
            for j in range(_L):
                r = g * _L + j
                u = uvec[j]
                i = ivec[j]
                pltpu.async_copy(
                    ut_hbm.at[pl.ds(u, 1), :], urows.at[pl.ds(r, 1), :], sem
                )
                pltpu.async_copy(
                    it_hbm.at[pl.ds(i, 1), :], irows.at[pl.ds(r, 1), :], sem
                )
            return carry

        lax.fori_loop(0, _P // _L, fire, 0)

        pltpu.make_async_copy(ut_hbm.at[pl.ds(0, _P), :], urows, sem).wait()
        pltpu.make_async_copy(it_hbm.at[pl.ds(0, _P), :], irows, sem).wait()

        def group(g, carry):
            rows = g * _L + lanes
            acc = jnp.zeros((_L,), jnp.float32)
            for c in range(_D):
                cols = jnp.full((_L,), c, jnp.int32)
                ug = plsc.load_gather(urows, [rows, cols])
                vg = plsc.load_gather(irows, [rows, cols])
                acc = acc + ug * vg
            outv[pl.ds(p0 + g * _L, _L)] = acc
            return carry

        lax.fori_loop(0, _P // _L, group, 0)

    pltpu.sync_copy(outv, out_hbm.at[pl.ds(base, _BPW)])


def kernel(user, item, user_table, item_table):
    return _mf_kernel(user, item, user_table, item_table)


# speedup vs baseline: 1.5624x; 1.0035x over previous
import functools

import jax
import jax.numpy as jnp
from jax import lax
from jax.experimental import pallas as pl
from jax.experimental.pallas import tpu as pltpu
from jax.experimental.pallas import tpu_sc as plsc

_info = plsc.get_sparse_core_info()
_NC, _NS, _L = _info.num_cores, _info.num_subcores, _info.num_lanes
_NW = _NC * _NS

_B = 16384
_D = 64
_BPW = _B // _NW
_P = 128
_NPASS = _BPW // _P
_NSEM = 4

_mesh = plsc.VectorSubcoreMesh(core_axis_name="c", subcore_axis_name="s")


@functools.partial(
    pl.kernel,
    out_type=jax.ShapeDtypeStruct((_B,), jnp.float32),
    mesh=_mesh,
    compiler_params=pltpu.CompilerParams(needs_layout_passes=False),
    scratch_types=[
        pltpu.VMEM((_BPW,), jnp.int32),
        pltpu.VMEM((_BPW,), jnp.int32),
        pltpu.VMEM((_P, _D), jnp.float32),
        pltpu.VMEM((_P, _D), jnp.float32),
        pltpu.VMEM((_P, _D), jnp.float32),
        pltpu.VMEM((_P, _D), jnp.float32),
        pltpu.VMEM((_BPW,), jnp.float32),
        pltpu.SemaphoreType.DMA((2 * _NSEM,)),
    ],
)
def _mf_kernel(user_hbm, item_hbm, ut_hbm, it_hbm, out_hbm,
               uidxv, iidxv, urows0, irows0, urows1, irows1, outv, sems):
    wid = lax.axis_index("s") * _NC + lax.axis_index("c")
    base = wid * _BPW

    pltpu.sync_copy(user_hbm.at[pl.ds(base, _BPW)], uidxv)
    pltpu.sync_copy(item_hbm.at[pl.ds(base, _BPW)], iidxv)

    ubufs = (urows0, urows1)
    ibufs = (irows0, irows1)
    lanes = lax.iota(jnp.int32, _L)

    def fire(p):
        bank = (p % 2) * _NSEM
        urows = ubufs[p % 2]
        irows = ibufs[p % 2]
        p0 = p * _P

        def body(g, carry):
            uvec = uidxv[pl.ds(p0 + g * _L, _L)]
            ivec = iidxv[pl.ds(p0 + g * _L, _L)]
            for j in range(_L):
                r = g * _L + j
                sem = sems.at[bank + j % _NSEM]
                u = uvec[j]
                i = ivec[j]
                pltpu.async_copy(
                    ut_hbm.at[pl.ds(u, 1), :], urows.at[pl.ds(r, 1), :], sem
                )
                pltpu.async_copy(
                    it_hbm.at[pl.ds(i, 1), :], irows.at[pl.ds(r, 1), :], sem
                )
            return carry

        lax.fori_loop(0, _P // _L, body, 0)

    def drain(p):
        bank = (p % 2) * _NSEM
        nrows = 2 * _P // _NSEM
        for k in range(_NSEM):
            pltpu.make_async_copy(
                ut_hbm.at[pl.ds(0, nrows), :],
                ubufs[p % 2].at[pl.ds(0, nrows), :],
                sems.at[bank + k],
            ).wait()

    def compute(p):
        urows = ubufs[p % 2]
        irows = ibufs[p % 2]
        p0 = p * _P

        def group(g, carry):
            rows = g * _L + lanes
            acc = jnp.zeros((_L,), jnp.float32)
            for c in range(_D):
                cols = jnp.full((_L,), c, jnp.int32)
                ug = plsc.load_gather(urows, [rows, cols])
                vg = plsc.load_gather(irows, [rows, cols])
                acc = acc + ug * vg
            outv[pl.ds(p0 + g * _L, _L)] = acc
            return carry

        lax.fori_loop(0, _P // _L, group, 0)

    fire(0)
    for p in range(1, _NPASS):
        fire(p)
        drain(p - 1)
        compute(p - 1)
    drain(_NPASS - 1)
    compute(_NPASS - 1)

    pltpu.sync_copy(outv, out_hbm.at[pl.ds(base, _BPW)])


def kernel(user, item, user_table, item_table):
    return _mf_kernel(user, item, user_table, item_table)
